# baseline (speedup 1.0000x reference)
import functools

import jax
import jax.numpy as jnp
from jax.experimental import pallas as pl
from jax.experimental.pallas import tpu as pltpu

_NH = 2
_RW = 14 // _NH


def _fused_kernel(x_ref, w_ref, b_ref, o_ref):
    u = x_ref[0].astype(jnp.bfloat16)
    u = jnp.transpose(u, (1, 0, 2, 3)).reshape(_RW, 48, 224)
    t = jnp.transpose(u, (0, 2, 1)).reshape(_RW, 14, 16, 48)
    m = _RW * 14
    acc = jnp.zeros((m, 768), jnp.float32)
    for kw in range(16):
        s = t[:, :, kw, :].reshape(m, 48)
        acc += jnp.dot(s, w_ref[kw], preferred_element_type=jnp.float32)
    o_ref[0] = acc + b_ref[...]


@jax.jit
def _patch_embed(x, W, b):
    B = x.shape[0]
    xv = x.reshape(B, 3, 14, 16, 224)
    w = W.transpose(3, 1, 2, 0).reshape(16, 48, 768).astype(jnp.bfloat16)
    bias = b.reshape(1, 768)
    m = _RW * 14
    out = pl.pallas_call(
        _fused_kernel,
        grid=(B, _NH),
        in_specs=[
            pl.BlockSpec((1, 3, _RW, 16, 224), lambda i, j: (i, 0, j, 0, 0)),
            pl.BlockSpec((16, 48, 768), lambda i, j: (0, 0, 0)),
            pl.BlockSpec((1, 768), lambda i, j: (0, 0)),
        ],
        out_specs=pl.BlockSpec((1, m, 768), lambda i, j: (i * _NH + j, 0, 0)),
        out_shape=jax.ShapeDtypeStruct((B * _NH, m, 768), jnp.float32),
        compiler_params=pltpu.CompilerParams(
            dimension_semantics=("parallel", "arbitrary")
        ),
    )(xv, w, bias)
    return out.reshape(B, 196, 768)


def kernel(x, W, b):
    return _patch_embed(x, W, b)


# speedup vs baseline: 1.7076x; 1.7076x over previous
import functools

import jax
import jax.numpy as jnp
from jax.experimental import pallas as pl
from jax.experimental.pallas import tpu as pltpu

_BI = 2


def _fused_kernel(x_ref, w_ref, b_ref, o_ref):
    u = x_ref[...].astype(jnp.bfloat16)
    u = jnp.transpose(u, (0, 2, 1, 3, 4)).reshape(_BI * 14, 48, 224)
    t = jnp.transpose(u, (0, 2, 1)).reshape(_BI * 14, 14, 16, 48)
    m = _BI * 196
    acc = jnp.zeros((m, 768), jnp.float32)
    for kw in range(16):
        s = t[:, :, kw, :].reshape(m, 48)
        acc += jnp.dot(s, w_ref[kw], preferred_element_type=jnp.float32)
    o_ref[...] = (acc + b_ref[...]).reshape(_BI, 196, 768)


@jax.jit
def _patch_embed(x, W, b):
    B = x.shape[0]
    xv = x.reshape(B, 3, 14, 16, 224)
    w = W.transpose(3, 1, 2, 0).reshape(16, 48, 768).astype(jnp.bfloat16)
    bias = b.reshape(1, 768)
    out = pl.pallas_call(
        _fused_kernel,
        grid=(B // _BI,),
        in_specs=[
            pl.BlockSpec((_BI, 3, 14, 16, 224), lambda i: (i, 0, 0, 0, 0)),
            pl.BlockSpec((16, 48, 768), lambda i: (0, 0, 0)),
            pl.BlockSpec((1, 768), lambda i: (0, 0)),
        ],
        out_specs=pl.BlockSpec((_BI, 196, 768), lambda i: (i, 0, 0)),
        out_shape=jax.ShapeDtypeStruct((B, 196, 768), jnp.float32),
        compiler_params=pltpu.CompilerParams(
            dimension_semantics=("parallel",)
        ),
    )(xv, w, bias)
    return out


def kernel(x, W, b):
    return _patch_embed(x, W, b)
